# baseline (speedup 1.0000x reference)
import jax
import jax.numpy as jnp
from jax import lax
from jax.experimental import pallas as pl
from jax.experimental.pallas import tpu as pltpu
from jax.experimental.pallas import tpu_sc as plsc

_C1 = 384
_C2 = 192
_NB = 4
_HW = 224 * 224
_BC = _NB * _C1
_RB = 128
_CB = 3584
_NR = _BC // _RB
_NCOL = _HW // _CB
_NW = 32
_SPL = 7
_CHW = _HW // _SPL
_NROWS = _NB * _C2 * _SPL
_RPW = _NROWS // _NW
_G = 8
_NGRP = _RPW // _G


def _mean_body(x_ref, o_ref):
    j = pl.program_id(1)
    part = jnp.sum(x_ref[...], axis=1, keepdims=True)
    part = part.reshape(1, _RB, 1)

    @pl.when(j == 0)
    def _():
        o_ref[...] = part

    @pl.when(j != 0)
    def _():
        o_ref[...] = o_ref[...] + part


def _scores_body(s_ref, w1_ref, w2_ref, o_ref):
    m = s_ref[...] * (1.0 / _HW)
    h = lax.dot_general(m, w1_ref[...], (((1,), (1,)), ((), ())),
                        preferred_element_type=jnp.float32)
    h = jnp.maximum(h, 0.0)
    z = lax.dot_general(h, w2_ref[...], (((1,), (1,)), ((), ())),
                        preferred_element_type=jnp.float32)
    y = jax.nn.sigmoid(z)

    yi = lax.broadcast_in_dim(y, (_NB, _C1, _C1), (0, 1))
    yj = lax.broadcast_in_dim(y, (_NB, _C1, _C1), (0, 2))
    jidx = lax.broadcasted_iota(jnp.int32, (_NB, _C1, _C1), 2)
    iidx = lax.broadcasted_iota(jnp.int32, (_NB, _C1, _C1), 1)
    beats = (yj > yi) | ((yj == yi) & (jidx < iidx))
    rank = jnp.sum(beats.astype(jnp.int32), axis=2)

    rnk3 = lax.broadcast_in_dim(rank, (_NB, _C1, _C1), (0, 1))
    pidx = lax.broadcasted_iota(jnp.int32, (_NB, _C1, _C1), 2)
    chan = lax.broadcasted_iota(jnp.int32, (_NB, _C1, _C1), 1)
    sel = jnp.sum(jnp.where(rnk3 == pidx, chan, 0), axis=1)
    boff = lax.broadcasted_iota(jnp.int32, (_NB, _C1), 0) * _C1
    o_ref[...] = sel + boff


def _gather_body(x_hbm, gi_hbm, o_hbm, idx_v, buf0, buf1, sem0, sem1):
    wid = lax.axis_index("s") * 2 + lax.axis_index("c")
    base = wid * _RPW
    pltpu.sync_copy(gi_hbm.at[pl.ds(base, _RPW)], idx_v)
    bufs = (buf0, buf1)
    sems = (sem0, sem1)
    cps = [None, None]
    cps[0] = pltpu.async_copy(x_hbm.at[idx_v.at[pl.ds(0, _G)]], buf0, sem0)
    for t in range(_NGRP):
        cur = t % 2
        if t + 1 < _NGRP:
            nxt = (t + 1) % 2
            cps[nxt] = pltpu.async_copy(
                x_hbm.at[idx_v.at[pl.ds((t + 1) * _G, _G)]], bufs[nxt], sems[nxt])
        cps[cur].wait()
        pltpu.sync_copy(bufs[cur], o_hbm.at[pl.ds(base + t * _G, _G)])


def kernel(x, W1, W2):
    b, c, h, w = x.shape
    x2 = x.reshape(_BC, _HW)

    sums = pl.pallas_call(
        _mean_body,
        grid=(_NR, _NCOL),
        in_specs=[pl.BlockSpec((_RB, _CB), lambda i, j: (i, j))],
        out_specs=pl.BlockSpec((1, _RB, 1), lambda i, j: (i, 0, 0)),
        out_shape=jax.ShapeDtypeStruct((_NR, _RB, 1), jnp.float32),
        compiler_params=pltpu.CompilerParams(
            dimension_semantics=("parallel", "arbitrary")),
    )(x2)

    flat_idx = pl.pallas_call(
        _scores_body,
        out_shape=jax.ShapeDtypeStruct((_NB, _C1), jnp.int32),
    )(sums.reshape(_NB, _C1), W1, W2)

    planes = flat_idx[:, :_C2].reshape(_NB * _C2)
    gidx = (planes[:, None] * _SPL
            + jnp.arange(_SPL, dtype=jnp.int32)[None, :]).reshape(_NROWS)

    mesh = plsc.VectorSubcoreMesh(core_axis_name="c", subcore_axis_name="s")
    gathered = pl.kernel(
        _gather_body,
        out_type=jax.ShapeDtypeStruct((_NROWS, _CHW), jnp.float32),
        mesh=mesh,
        scratch_types=[
            pltpu.VMEM((_RPW,), jnp.int32),
            pltpu.VMEM((_G, _CHW), jnp.float32),
            pltpu.VMEM((_G, _CHW), jnp.float32),
            pltpu.SemaphoreType.DMA,
            pltpu.SemaphoreType.DMA,
        ],
    )(x2.reshape(_BC * _SPL, _CHW), gidx)

    return gathered.reshape(b, _C2, h, w)


# speedup vs baseline: 1.6530x; 1.6530x over previous
import jax
import jax.numpy as jnp
from jax import lax
from jax.experimental import pallas as pl
from jax.experimental.pallas import tpu as pltpu
from jax.experimental.pallas import tpu_sc as plsc

_C1 = 384
_C2 = 192
_NB = 4
_H = 224
_W = 224
_BC = _NB * _C1
_TCP = 768
_SCP = _BC - _TCP
_PB = 32
_NP = _TCP // _PB
_NW = 32
_PPW = _SCP // _NW
_VL = 16
_NCH = _W // _VL


def _sc_mean_body(x_hbm, o_hbm, buf0, buf1, acc, obuf, sem0, sem1):
    wid = lax.axis_index("s") * 2 + lax.axis_index("c")
    base = _TCP + wid * _PPW
    bufs = (buf0, buf1)
    sems = (sem0, sem1)
    cps = [None, None]
    cps[0] = pltpu.async_copy(x_hbm.at[pl.ds(base, 1)], buf0, sem0)
    for t in range(_PPW):
        cur = t % 2
        if t + 1 < _PPW:
            nxt = (t + 1) % 2
            cps[nxt] = pltpu.async_copy(
                x_hbm.at[pl.ds(base + t + 1, 1)], bufs[nxt], sems[nxt])
        cps[cur].wait()
        b = bufs[cur]
        acc[...] = jnp.zeros((_VL,), jnp.float32)

        def _row(i, carry):
            v = acc[...]
            for cix in range(_NCH):
                v = v + b[0, i, pl.ds(cix * _VL, _VL)]
            acc[...] = v
            return carry

        lax.fori_loop(0, _H, _row, 0)
        obuf[t, :] = acc[...]
    pltpu.sync_copy(obuf, o_hbm.at[pl.ds(wid * _PPW, _PPW)])


def _tc_mean_body(x_ref, o_ref):
    s = jnp.sum(x_ref[...], axis=2)
    o_ref[...] = jnp.sum(s, axis=1).reshape(_PB, 1)


def _scores_body(s_ref, w1_ref, w2_ref, o_ref):
    m = s_ref[...] * (1.0 / (_H * _W))
    h = lax.dot_general(m, w1_ref[...], (((1,), (1,)), ((), ())),
                        preferred_element_type=jnp.float32)
    h = jnp.maximum(h, 0.0)
    z = lax.dot_general(h, w2_ref[...], (((1,), (1,)), ((), ())),
                        preferred_element_type=jnp.float32)
    y = jax.nn.sigmoid(z)

    yi = lax.broadcast_in_dim(y, (_NB, _C1, _C1), (0, 1))
    yj = lax.broadcast_in_dim(y, (_NB, _C1, _C1), (0, 2))
    jidx = lax.broadcasted_iota(jnp.int32, (_NB, _C1, _C1), 2)
    iidx = lax.broadcasted_iota(jnp.int32, (_NB, _C1, _C1), 1)
    beats = (yj > yi) | ((yj == yi) & (jidx < iidx))
    rank = jnp.sum(beats.astype(jnp.int32), axis=2)

    rnk3 = lax.broadcast_in_dim(rank, (_NB, _C1, _C1), (0, 1))
    pidx = lax.broadcasted_iota(jnp.int32, (_NB, _C1, _C1), 2)
    chan = lax.broadcasted_iota(jnp.int32, (_NB, _C1, _C1), 1)
    sel = jnp.sum(jnp.where(rnk3 == pidx, chan, 0), axis=1)
    boff = lax.broadcasted_iota(jnp.int32, (_NB, _C1), 0) * _C1
    o_ref[...] = sel + boff


def _gather_body(idx_ref, x_ref, o_ref):
    o_ref[...] = x_ref[...]


def kernel(x, W1, W2):
    b, c, h, w = x.shape
    x3 = x.reshape(_BC, _H, _W)

    mesh = plsc.VectorSubcoreMesh(core_axis_name="c", subcore_axis_name="s")
    sc_part = pl.kernel(
        _sc_mean_body,
        out_type=jax.ShapeDtypeStruct((_SCP, _VL), jnp.float32),
        mesh=mesh,
        scratch_types=[
            pltpu.VMEM((1, _H, _W), jnp.float32),
            pltpu.VMEM((1, _H, _W), jnp.float32),
            pltpu.VMEM((_VL,), jnp.float32),
            pltpu.VMEM((_PPW, _VL), jnp.float32),
            pltpu.SemaphoreType.DMA,
            pltpu.SemaphoreType.DMA,
        ],
    )(x3)

    tc_sums = pl.pallas_call(
        _tc_mean_body,
        grid=(_NP,),
        in_specs=[pl.BlockSpec((_PB, _H, _W), lambda i: (i, 0, 0))],
        out_specs=pl.BlockSpec((_PB, 1), lambda i: (i, 0)),
        out_shape=jax.ShapeDtypeStruct((_TCP, 1), jnp.float32),
        compiler_params=pltpu.CompilerParams(
            dimension_semantics=("arbitrary",)),
    )(x3)

    sums = jnp.concatenate(
        [tc_sums.reshape(_TCP), jnp.sum(sc_part, axis=1)]).reshape(_NB, _C1)

    flat_idx = pl.pallas_call(
        _scores_body,
        out_shape=jax.ShapeDtypeStruct((_NB, _C1), jnp.int32),
    )(sums, W1, W2)

    planes = flat_idx[:, :_C2].reshape(_NB * _C2)

    gathered = pl.pallas_call(
        _gather_body,
        grid_spec=pltpu.PrefetchScalarGridSpec(
            num_scalar_prefetch=1,
            grid=(_NB * _C2,),
            in_specs=[pl.BlockSpec((1, _H, _W),
                                   lambda i, idx_ref: (idx_ref[i], 0, 0))],
            out_specs=pl.BlockSpec((1, _H, _W),
                                   lambda i, idx_ref: (i, 0, 0)),
        ),
        out_shape=jax.ShapeDtypeStruct((_NB * _C2, _H, _W), jnp.float32),
    )(planes, x3)

    return gathered.reshape(b, _C2, h, w)
